# Initial kernel scaffold; baseline (speedup 1.0000x reference)
import functools

import jax
import jax.numpy as jnp
from jax import lax
from jax.experimental import pallas as pl
from jax.experimental.pallas import tpu as pltpu
from jax.experimental.pallas import tpu_sc as plsc

N = 10000
E = 320000
F = 128
H = 20
C = 10

HP = 32
NP = 10240
NC, NS = 2, 16
NW = NC * NS
G = 125
NB = 8
CH = G * NB
EPW = E // NW
NSC = EPW // CH
RPT = NP // NS

_mesh = plsc.VectorSubcoreMesh(core_axis_name="c", subcore_axis_name="s")
_f32 = jnp.float32


def _zero_fill_2d(buf, nrows):
    z = jnp.zeros((16,), _f32)

    def body(i, carry):
        buf[i, pl.ds(0, 16)] = z
        buf[i, pl.ds(16, 16)] = z
        return carry

    lax.fori_loop(0, nrows, body, 0, unroll=8)


@functools.partial(
    pl.kernel,
    out_type=jax.ShapeDtypeStruct((NC, NP), _f32),
    mesh=_mesh,
    scratch_types=[
        pltpu.VMEM_SHARED((NP,), _f32),
        pltpu.VMEM((NB, G), jnp.int32),
        pltpu.VMEM((NB, G), _f32),
        pltpu.VMEM((RPT,), _f32),
    ],
)
def _deg_kernel(col_hbm, ew_hbm, out_hbm, acc, coli, eww, zbuf):
    c = lax.axis_index("c")
    s = lax.axis_index("s")
    wid = c * NS + s

    z = jnp.zeros((16,), _f32)

    def zb(i, carry):
        zbuf[pl.ds(i * 16, 16)] = z
        return carry

    lax.fori_loop(0, RPT // 16, zb, 0, unroll=8)
    pltpu.sync_copy(zbuf, acc.at[pl.ds(s * RPT, RPT)])
    plsc.subcore_barrier()

    def chunk(k, carry):
        gbase = wid * (EPW // G) + k * NB
        pltpu.sync_copy(col_hbm.at[pl.ds(gbase, NB)], coli)
        pltpu.sync_copy(ew_hbm.at[pl.ds(gbase, NB)], eww)
        for j in range(NB):
            pltpu.sync_copy(eww.at[j], acc.at[coli.at[j]], add=True)
        return carry

    lax.fori_loop(0, NSC, chunk, 0)
    plsc.subcore_barrier()
    pltpu.sync_copy(acc.at[pl.ds(s * RPT, RPT)], out_hbm.at[c, pl.ds(s * RPT, RPT)])


@functools.partial(
    pl.kernel,
    out_type=jax.ShapeDtypeStruct((NC, NP, HP), _f32),
    mesh=_mesh,
    scratch_types=[
        pltpu.VMEM_SHARED((NP, HP), _f32),
        pltpu.VMEM((NB, G), jnp.int32),
        pltpu.VMEM((NB, G), jnp.int32),
        pltpu.VMEM((CH,), _f32),
        pltpu.VMEM((CH, HP), _f32),
        pltpu.VMEM((128, HP), _f32),
        pltpu.SemaphoreType.DMA,
        pltpu.SemaphoreType.DMA,
    ],
)
def _agg_kernel(hp_hbm, row_hbm, col_hbm, ew_hbm, out_hbm,
                acc, rowi, coli, eww, rows, zbuf, gsem, ssem):
    c = lax.axis_index("c")
    s = lax.axis_index("s")
    wid = c * NS + s

    _zero_fill_2d(zbuf, 128)
    for q in range(RPT // 128):
        pltpu.sync_copy(zbuf, acc.at[pl.ds(s * RPT + q * 128, 128)])
    plsc.subcore_barrier()

    def chunk(k, carry):
        gbase = wid * (EPW // G) + k * NB
        ebase = wid * EPW + k * CH
        pltpu.sync_copy(row_hbm.at[pl.ds(gbase, NB)], rowi)
        pltpu.sync_copy(col_hbm.at[pl.ds(gbase, NB)], coli)
        pltpu.sync_copy(ew_hbm.at[pl.ds(ebase, CH)], eww)
        gcps = [
            pltpu.async_copy(hp_hbm.at[rowi.at[j]],
                             rows.at[pl.ds(j * G, G)], gsem)
            for j in range(NB)
        ]
        for cp in gcps:
            cp.wait()

        def mul(i, carry2):
            w = jnp.full((16,), eww[i])
            rows[i, pl.ds(0, 16)] = rows[i, pl.ds(0, 16)] * w
            rows[i, pl.ds(16, 16)] = rows[i, pl.ds(16, 16)] * w
            return carry2

        lax.fori_loop(0, CH, mul, 0, unroll=8)

        scps = [
            pltpu.async_copy(rows.at[pl.ds(j * G, G)],
                             acc.at[coli.at[j]], ssem, add=True)
            for j in range(NB)
        ]
        for cp in scps:
            cp.wait()
        return carry

    lax.fori_loop(0, NSC, chunk, 0)
    plsc.subcore_barrier()
    for q in range(RPT // 128):
        pltpu.sync_copy(acc.at[pl.ds(s * RPT + q * 128, 128)],
                        out_hbm.at[c, pl.ds(s * RPT + q * 128, 128)])


BN = 1000
GRID = N // BN


def _dis_from_degt(degt):
    deg = degt[:, 0:1] + degt[:, 1:2] + 1.0
    return lax.rsqrt(deg)


def _prep_body(x_ref, w_ref, degt_ref, out_ref):
    dis = _dis_from_degt(degt_ref[...])
    h = jnp.dot(x_ref[...], w_ref[...], preferred_element_type=_f32)
    out_ref[...] = h * dis


def _layer_body(s_ref, hp_ref, degt_ref, b_ref, w_ref, out_ref):
    dis = _dis_from_degt(degt_ref[...])
    o = dis * (s_ref[0] + s_ref[1] + hp_ref[...]) + b_ref[...]
    s2 = jnp.sum(o * o, axis=1, keepdims=True)
    z = jnp.maximum(o / jnp.maximum(jnp.sqrt(s2), 1e-12), 0.0)
    out_ref[...] = dis * jnp.dot(z, w_ref[...], preferred_element_type=_f32)


def _final_body(s_ref, hp_ref, degt_ref, b_ref, lw_ref, lb_ref, out_ref, pool):
    i = pl.program_id(0)
    dis = _dis_from_degt(degt_ref[...])
    o = dis * (s_ref[0] + s_ref[1] + hp_ref[...]) + b_ref[...]
    s2 = jnp.sum(o * o, axis=1, keepdims=True)
    z = jnp.maximum(o / jnp.maximum(jnp.sqrt(s2), 1e-12), 0.0)
    bm = jnp.max(z, axis=0, keepdims=True)

    @pl.when(i == 0)
    def _():
        pool[...] = bm

    @pl.when(i > 0)
    def _():
        pool[...] = jnp.maximum(pool[...], bm)

    @pl.when(i == GRID - 1)
    def _():
        out_ref[...] = (
            jnp.dot(pool[...], lw_ref[...], preferred_element_type=_f32)
            + lb_ref[...]
        )


def _prep_call(x, w1p, degt):
    return pl.pallas_call(
        _prep_body,
        grid=(GRID,),
        in_specs=[
            pl.BlockSpec((BN, F), lambda i: (i, 0)),
            pl.BlockSpec((F, HP), lambda i: (0, 0)),
            pl.BlockSpec((BN, NC), lambda i: (i, 0)),
        ],
        out_specs=pl.BlockSpec((BN, HP), lambda i: (i, 0)),
        out_shape=jax.ShapeDtypeStruct((NP, HP), _f32),
    )(x, w1p, degt)


def _layer_call(S, hp, degt, bp, wp):
    return pl.pallas_call(
        _layer_body,
        grid=(GRID,),
        in_specs=[
            pl.BlockSpec((NC, BN, HP), lambda i: (0, i, 0)),
            pl.BlockSpec((BN, HP), lambda i: (i, 0)),
            pl.BlockSpec((BN, NC), lambda i: (i, 0)),
            pl.BlockSpec((1, HP), lambda i: (0, 0)),
            pl.BlockSpec((HP, HP), lambda i: (0, 0)),
        ],
        out_specs=pl.BlockSpec((BN, HP), lambda i: (i, 0)),
        out_shape=jax.ShapeDtypeStruct((NP, HP), _f32),
    )(S, hp, degt, bp, wp)


def _final_call(S, hp, degt, bp, lwp, lbp):
    return pl.pallas_call(
        _final_body,
        grid=(GRID,),
        in_specs=[
            pl.BlockSpec((NC, BN, HP), lambda i: (0, i, 0)),
            pl.BlockSpec((BN, HP), lambda i: (i, 0)),
            pl.BlockSpec((BN, NC), lambda i: (i, 0)),
            pl.BlockSpec((1, HP), lambda i: (0, 0)),
            pl.BlockSpec((HP, C), lambda i: (0, 0)),
            pl.BlockSpec((1, C), lambda i: (0, 0)),
        ],
        out_specs=pl.BlockSpec((1, C), lambda i: (0, 0)),
        out_shape=jax.ShapeDtypeStruct((1, C), _f32),
        scratch_shapes=[pltpu.VMEM((1, HP), _f32)],
    )(S, hp, degt, bp, lwp, lbp)


def kernel(x, edge_index, edge_weights, batch,
           conv1_w, conv1_b, conv2_w, conv2_b, conv3_w, conv3_b,
           lin_w, lin_b):
    del batch
    x = x.astype(_f32)
    row_g = edge_index[0].reshape(E // G, G)
    col_g = edge_index[1].reshape(E // G, G)
    ew_g = edge_weights.reshape(E // G, G)

    pad_h = HP - H
    w1p = jnp.pad(conv1_w, ((0, 0), (0, pad_h)))
    w2p = jnp.pad(conv2_w, ((0, pad_h), (0, pad_h)))
    w3p = jnp.pad(conv3_w, ((0, pad_h), (0, pad_h)))
    b1p = jnp.pad(conv1_b, (0, pad_h)).reshape(1, HP)
    b2p = jnp.pad(conv2_b, (0, pad_h)).reshape(1, HP)
    b3p = jnp.pad(conv3_b, (0, pad_h)).reshape(1, HP)
    lwp = jnp.pad(lin_w, ((0, pad_h), (0, 0)))
    lbp = lin_b.reshape(1, C)

    degp = _deg_kernel(col_g, ew_g)
    degt = degp.T

    hp1 = _prep_call(x, w1p, degt)
    S1 = _agg_kernel(hp1, row_g, col_g, edge_weights)
    hp2 = _layer_call(S1, hp1, degt, b1p, w2p)
    S2 = _agg_kernel(hp2, row_g, col_g, edge_weights)
    hp3 = _layer_call(S2, hp2, degt, b2p, w3p)
    S3 = _agg_kernel(hp3, row_g, col_g, edge_weights)
    return _final_call(S3, hp3, degt, b3p, lwp, lbp)


# speedup vs baseline: 35.8170x; 35.8170x over previous
import functools

import jax
import jax.numpy as jnp
from jax import lax
from jax.experimental import pallas as pl
from jax.experimental.pallas import tpu as pltpu
from jax.experimental.pallas import tpu_sc as plsc

N = 10000
E = 320000
F = 128
H = 20
C = 10

HP = 32
NP = 10240
NC, NS = 2, 16
NW = NC * NS
G = 125
NB = 16
CH = G * NB
EPW = E // NW
NSC = EPW // CH
RPT = NP // NS

_mesh = plsc.VectorSubcoreMesh(core_axis_name="c", subcore_axis_name="s")
_f32 = jnp.float32
_sc_params = pltpu.CompilerParams(use_tc_tiling_on_sc=False)


def _zero_fill_2d(buf, nrows):
    z = jnp.zeros((16,), _f32)

    def body(i, carry):
        buf[i, pl.ds(0, 16)] = z
        buf[i, pl.ds(16, 16)] = z
        return carry

    lax.fori_loop(0, nrows, body, 0, unroll=8)


@functools.partial(
    pl.kernel,
    out_type=jax.ShapeDtypeStruct((NC, NP), _f32),
    mesh=_mesh,
    compiler_params=_sc_params,
    scratch_types=[
        pltpu.VMEM_SHARED((NP,), _f32),
        pltpu.VMEM((NB, G), jnp.int32),
        pltpu.VMEM((NB, G), _f32),
        pltpu.VMEM((RPT,), _f32),
    ],
)
def _deg_kernel(col_hbm, ew_hbm, out_hbm, acc, coli, eww, zbuf):
    c = lax.axis_index("c")
    s = lax.axis_index("s")
    wid = c * NS + s

    z = jnp.zeros((16,), _f32)

    def zb(i, carry):
        zbuf[pl.ds(i * 16, 16)] = z
        return carry

    lax.fori_loop(0, RPT // 16, zb, 0, unroll=8)
    pltpu.sync_copy(zbuf, acc.at[pl.ds(s * RPT, RPT)])
    plsc.subcore_barrier()

    def chunk(k, carry):
        gbase = wid * (EPW // G) + k * NB
        pltpu.sync_copy(col_hbm.at[pl.ds(gbase, NB)], coli)
        pltpu.sync_copy(ew_hbm.at[pl.ds(gbase, NB)], eww)
        for j in range(NB):
            pltpu.sync_copy(eww.at[j], acc.at[coli.at[j]], add=True)
        return carry

    lax.fori_loop(0, NSC, chunk, 0)
    plsc.subcore_barrier()
    pltpu.sync_copy(acc.at[pl.ds(s * RPT, RPT)], out_hbm.at[c, pl.ds(s * RPT, RPT)])


@functools.partial(
    pl.kernel,
    out_type=jax.ShapeDtypeStruct((NC, NP, HP), _f32),
    mesh=_mesh,
    compiler_params=_sc_params,
    scratch_types=[
        pltpu.VMEM_SHARED((NP, HP), _f32),
        pltpu.VMEM((NB, G), jnp.int32),
        pltpu.VMEM((NB, G), jnp.int32),
        pltpu.VMEM((CH,), _f32),
        pltpu.VMEM((CH, HP), _f32),
        pltpu.VMEM((128, HP), _f32),
        pltpu.SemaphoreType.DMA,
        pltpu.SemaphoreType.DMA,
    ],
)
def _agg_kernel(hp_hbm, row_hbm, col_hbm, ew_hbm, out_hbm,
                acc, rowi, coli, eww, rows, zbuf, gsem, ssem):
    c = lax.axis_index("c")
    s = lax.axis_index("s")
    wid = c * NS + s

    _zero_fill_2d(zbuf, 128)
    for q in range(RPT // 128):
        pltpu.sync_copy(zbuf, acc.at[pl.ds(s * RPT + q * 128, 128)])
    plsc.subcore_barrier()

    def chunk(k, carry):
        gbase = wid * (EPW // G) + k * NB
        ebase = wid * EPW + k * CH
        pltpu.sync_copy(row_hbm.at[pl.ds(gbase, NB)], rowi)
        pltpu.sync_copy(col_hbm.at[pl.ds(gbase, NB)], coli)
        pltpu.sync_copy(ew_hbm.at[pl.ds(ebase, CH)], eww)
        gcps = [
            pltpu.async_copy(hp_hbm.at[rowi.at[j]],
                             rows.at[pl.ds(j * G, G)], gsem)
            for j in range(NB)
        ]
        for cp in gcps:
            cp.wait()

        def mul(i, carry2):
            w16 = eww[pl.ds(i * 16, 16)]
            for l in range(16):
                e = i * 16 + l
                w = jnp.full((16,), w16[l])
                rows[e, pl.ds(0, 16)] = rows[e, pl.ds(0, 16)] * w
                rows[e, pl.ds(16, 16)] = rows[e, pl.ds(16, 16)] * w
            return carry2

        lax.fori_loop(0, CH // 16, mul, 0)

        scps = [
            pltpu.async_copy(rows.at[pl.ds(j * G, G)],
                             acc.at[coli.at[j]], ssem, add=True)
            for j in range(NB)
        ]
        for cp in scps:
            cp.wait()
        return carry

    lax.fori_loop(0, NSC, chunk, 0)
    plsc.subcore_barrier()
    for q in range(RPT // 128):
        pltpu.sync_copy(acc.at[pl.ds(s * RPT + q * 128, 128)],
                        out_hbm.at[c, pl.ds(s * RPT + q * 128, 128)])


BN = 1000
GRID = N // BN


def _dis_from_degt(degt):
    deg = degt[:, 0:1] + degt[:, 1:2] + 1.0
    return lax.rsqrt(deg)


def _prep_body(x_ref, w_ref, degt_ref, out_ref):
    dis = _dis_from_degt(degt_ref[...])
    h = jnp.dot(x_ref[...], w_ref[...], preferred_element_type=_f32)
    out_ref[...] = h * dis


def _layer_body(s_ref, hp_ref, degt_ref, b_ref, w_ref, out_ref):
    dis = _dis_from_degt(degt_ref[...])
    o = dis * (s_ref[0] + s_ref[1] + hp_ref[...]) + b_ref[...]
    s2 = jnp.sum(o * o, axis=1, keepdims=True)
    z = jnp.maximum(o / jnp.maximum(jnp.sqrt(s2), 1e-12), 0.0)
    out_ref[...] = dis * jnp.dot(z, w_ref[...], preferred_element_type=_f32)


def _final_body(s_ref, hp_ref, degt_ref, b_ref, lw_ref, lb_ref, out_ref, pool):
    i = pl.program_id(0)
    dis = _dis_from_degt(degt_ref[...])
    o = dis * (s_ref[0] + s_ref[1] + hp_ref[...]) + b_ref[...]
    s2 = jnp.sum(o * o, axis=1, keepdims=True)
    z = jnp.maximum(o / jnp.maximum(jnp.sqrt(s2), 1e-12), 0.0)
    bm = jnp.max(z, axis=0, keepdims=True)

    @pl.when(i == 0)
    def _():
        pool[...] = bm

    @pl.when(i > 0)
    def _():
        pool[...] = jnp.maximum(pool[...], bm)

    @pl.when(i == GRID - 1)
    def _():
        out_ref[...] = (
            jnp.dot(pool[...], lw_ref[...], preferred_element_type=_f32)
            + lb_ref[...]
        )


def _prep_call(x, w1p, degt):
    return pl.pallas_call(
        _prep_body,
        grid=(GRID,),
        in_specs=[
            pl.BlockSpec((BN, F), lambda i: (i, 0)),
            pl.BlockSpec((F, HP), lambda i: (0, 0)),
            pl.BlockSpec((BN, NC), lambda i: (i, 0)),
        ],
        out_specs=pl.BlockSpec((BN, HP), lambda i: (i, 0)),
        out_shape=jax.ShapeDtypeStruct((NP, HP), _f32),
    )(x, w1p, degt)


def _layer_call(S, hp, degt, bp, wp):
    return pl.pallas_call(
        _layer_body,
        grid=(GRID,),
        in_specs=[
            pl.BlockSpec((NC, BN, HP), lambda i: (0, i, 0)),
            pl.BlockSpec((BN, HP), lambda i: (i, 0)),
            pl.BlockSpec((BN, NC), lambda i: (i, 0)),
            pl.BlockSpec((1, HP), lambda i: (0, 0)),
            pl.BlockSpec((HP, HP), lambda i: (0, 0)),
        ],
        out_specs=pl.BlockSpec((BN, HP), lambda i: (i, 0)),
        out_shape=jax.ShapeDtypeStruct((NP, HP), _f32),
    )(S, hp, degt, bp, wp)


def _final_call(S, hp, degt, bp, lwp, lbp):
    return pl.pallas_call(
        _final_body,
        grid=(GRID,),
        in_specs=[
            pl.BlockSpec((NC, BN, HP), lambda i: (0, i, 0)),
            pl.BlockSpec((BN, HP), lambda i: (i, 0)),
            pl.BlockSpec((BN, NC), lambda i: (i, 0)),
            pl.BlockSpec((1, HP), lambda i: (0, 0)),
            pl.BlockSpec((HP, C), lambda i: (0, 0)),
            pl.BlockSpec((1, C), lambda i: (0, 0)),
        ],
        out_specs=pl.BlockSpec((1, C), lambda i: (0, 0)),
        out_shape=jax.ShapeDtypeStruct((1, C), _f32),
        scratch_shapes=[pltpu.VMEM((1, HP), _f32)],
    )(S, hp, degt, bp, lwp, lbp)


def kernel(x, edge_index, edge_weights, batch,
           conv1_w, conv1_b, conv2_w, conv2_b, conv3_w, conv3_b,
           lin_w, lin_b):
    del batch
    x = x.astype(_f32)
    row_g = edge_index[0].reshape(E // G, G)
    col_g = edge_index[1].reshape(E // G, G)
    ew_g = edge_weights.reshape(E // G, G)

    pad_h = HP - H
    w1p = jnp.pad(conv1_w, ((0, 0), (0, pad_h)))
    w2p = jnp.pad(conv2_w, ((0, pad_h), (0, pad_h)))
    w3p = jnp.pad(conv3_w, ((0, pad_h), (0, pad_h)))
    b1p = jnp.pad(conv1_b, (0, pad_h)).reshape(1, HP)
    b2p = jnp.pad(conv2_b, (0, pad_h)).reshape(1, HP)
    b3p = jnp.pad(conv3_b, (0, pad_h)).reshape(1, HP)
    lwp = jnp.pad(lin_w, ((0, pad_h), (0, 0)))
    lbp = lin_b.reshape(1, C)

    degp = _deg_kernel(col_g, ew_g)
    degt = degp.T

    hp1 = _prep_call(x, w1p, degt)
    S1 = _agg_kernel(hp1, row_g, col_g, edge_weights)
    hp2 = _layer_call(S1, hp1, degt, b1p, w2p)
    S2 = _agg_kernel(hp2, row_g, col_g, edge_weights)
    hp3 = _layer_call(S2, hp2, degt, b2p, w3p)
    S3 = _agg_kernel(hp3, row_g, col_g, edge_weights)
    return _final_call(S3, hp3, degt, b3p, lwp, lbp)


# speedup vs baseline: 45.1465x; 1.2605x over previous
import functools

import jax
import jax.numpy as jnp
from jax import lax
from jax.experimental import pallas as pl
from jax.experimental.pallas import tpu as pltpu
from jax.experimental.pallas import tpu_sc as plsc

N = 10000
E = 320000
F = 128
H = 20
C = 10

HP = 32
NP = 10240
NC, NS = 2, 16
NW = NC * NS
G = 80
EPW = E // NW
GPW = EPW // G
CPC = 10
CH = G * CPC
NFULL = 12
TAILG = GPW - NFULL * CPC
RPT = NP // NS

_mesh = plsc.VectorSubcoreMesh(core_axis_name="c", subcore_axis_name="s")
_f32 = jnp.float32
_sc_params = pltpu.CompilerParams(use_tc_tiling_on_sc=False)


@functools.partial(
    pl.kernel,
    out_type=jax.ShapeDtypeStruct((NC, NP), _f32),
    mesh=_mesh,
    compiler_params=_sc_params,
    scratch_types=[
        pltpu.VMEM_SHARED((NP,), _f32),
        pltpu.VMEM((GPW, G), jnp.int32),
        pltpu.VMEM((GPW, G), _f32),
        pltpu.VMEM((RPT,), _f32),
        pltpu.SemaphoreType.DMA,
    ],
)
def _deg_kernel(col_hbm, ew_hbm, out_hbm, acc, coli, eww, zbuf, ssem):
    c = lax.axis_index("c")
    s = lax.axis_index("s")
    wid = c * NS + s

    pltpu.sync_copy(col_hbm.at[pl.ds(wid * GPW, GPW)], coli)
    pltpu.sync_copy(ew_hbm.at[pl.ds(wid * GPW, GPW)], eww)

    z = jnp.zeros((16,), _f32)

    def zb(i, carry):
        zbuf[pl.ds(i * 16, 16)] = z
        return carry

    lax.fori_loop(0, RPT // 16, zb, 0, unroll=8)
    pltpu.sync_copy(zbuf, acc.at[pl.ds(s * RPT, RPT)])
    plsc.subcore_barrier()

    for b in range(5):
        cps = [
            pltpu.async_copy(eww.at[b * 25 + j], acc.at[coli.at[b * 25 + j]],
                             ssem, add=True)
            for j in range(25)
        ]
        for cp in cps:
            cp.wait()
    plsc.subcore_barrier()
    pltpu.sync_copy(acc.at[pl.ds(s * RPT, RPT)], out_hbm.at[c, pl.ds(s * RPT, RPT)])


@functools.partial(
    pl.kernel,
    out_type=jax.ShapeDtypeStruct((NC, NP, HP), _f32),
    mesh=_mesh,
    compiler_params=_sc_params,
    scratch_types=[
        pltpu.VMEM_SHARED((NP, HP), _f32),
        pltpu.VMEM((EPW,), jnp.int32),
        pltpu.VMEM((GPW, G), jnp.int32),
        pltpu.VMEM((EPW,), _f32),
        pltpu.VMEM((2, CH, HP), _f32),
        pltpu.VMEM((128, HP), _f32),
        pltpu.SemaphoreType.DMA,
        pltpu.SemaphoreType.DMA,
    ],
)
def _agg_kernel(hp_hbm, row_hbm, col_hbm, ew_hbm, out_hbm,
                acc, rowi, coli, eww, rows, zbuf, gsem, ssem):
    c = lax.axis_index("c")
    s = lax.axis_index("s")
    wid = c * NS + s

    pltpu.sync_copy(row_hbm.at[pl.ds(wid * EPW, EPW)], rowi)
    pltpu.sync_copy(col_hbm.at[pl.ds(wid * GPW, GPW)], coli)
    pltpu.sync_copy(ew_hbm.at[pl.ds(wid * EPW, EPW)], eww)

    z = jnp.zeros((16,), _f32)

    def zb(i, carry):
        zbuf[i, pl.ds(0, 16)] = z
        zbuf[i, pl.ds(16, 16)] = z
        return carry

    lax.fori_loop(0, 128, zb, 0, unroll=8)
    for q in range(RPT // 128):
        pltpu.sync_copy(zbuf, acc.at[pl.ds(s * RPT + q * 128, 128)])
    plsc.subcore_barrier()

    ngrp = [CPC] * NFULL + [TAILG]

    def fire_gathers(k):
        slot = k % 2
        g0 = k * CPC
        return [
            pltpu.async_copy(
                hp_hbm.at[rowi.at[pl.ds((g0 + gi) * G, G)]],
                rows.at[slot, pl.ds(gi * G, G)],
                gsem)
            for gi in range(ngrp[k])
        ]

    def fire_scatters(k):
        slot = k % 2
        g0 = k * CPC
        return [
            pltpu.async_copy(
                rows.at[slot, pl.ds(gi * G, G)],
                acc.at[coli.at[g0 + gi]],
                ssem, add=True)
            for gi in range(ngrp[k])
        ]

    def mul(k):
        slot = k % 2
        e0 = k * CH

        def body(i, carry):
            w16 = eww[pl.ds(e0 + i * 16, 16)]
            for l in range(16):
                e = i * 16 + l
                w = jnp.full((16,), w16[l])
                rows[slot, e, pl.ds(0, 16)] = rows[slot, e, pl.ds(0, 16)] * w
                rows[slot, e, pl.ds(16, 16)] = rows[slot, e, pl.ds(16, 16)] * w
            return carry

        lax.fori_loop(0, ngrp[k] * G // 16, body, 0)

    nchunk = NFULL + 1
    gcps = fire_gathers(0)
    scps_prev = None
    for k in range(nchunk):
        for cp in gcps:
            cp.wait()
        gcps = []
        if scps_prev is not None:
            for cp in scps_prev:
                cp.wait()
        if k + 1 < nchunk:
            gcps = fire_gathers(k + 1)
        mul(k)
        scps_prev = fire_scatters(k)
    for cp in scps_prev:
        cp.wait()

    plsc.subcore_barrier()
    for q in range(RPT // 128):
        pltpu.sync_copy(acc.at[pl.ds(s * RPT + q * 128, 128)],
                        out_hbm.at[c, pl.ds(s * RPT + q * 128, 128)])


BN = 1000
GRID = N // BN


def _dis_from_degt(degt):
    deg = degt[:, 0:1] + degt[:, 1:2] + 1.0
    return lax.rsqrt(deg)


def _prep_body(x_ref, w_ref, degt_ref, out_ref):
    dis = _dis_from_degt(degt_ref[...])
    h = jnp.dot(x_ref[...], w_ref[...], preferred_element_type=_f32)
    out_ref[...] = h * dis


def _layer_body(s_ref, hp_ref, degt_ref, b_ref, w_ref, out_ref):
    dis = _dis_from_degt(degt_ref[...])
    o = dis * (s_ref[0] + s_ref[1] + hp_ref[...]) + b_ref[...]
    s2 = jnp.sum(o * o, axis=1, keepdims=True)
    z = jnp.maximum(o / jnp.maximum(jnp.sqrt(s2), 1e-12), 0.0)
    out_ref[...] = dis * jnp.dot(z, w_ref[...], preferred_element_type=_f32)


def _final_body(s_ref, hp_ref, degt_ref, b_ref, lw_ref, lb_ref, out_ref, pool):
    i = pl.program_id(0)
    dis = _dis_from_degt(degt_ref[...])
    o = dis * (s_ref[0] + s_ref[1] + hp_ref[...]) + b_ref[...]
    s2 = jnp.sum(o * o, axis=1, keepdims=True)
    z = jnp.maximum(o / jnp.maximum(jnp.sqrt(s2), 1e-12), 0.0)
    bm = jnp.max(z, axis=0, keepdims=True)

    @pl.when(i == 0)
    def _():
        pool[...] = bm

    @pl.when(i > 0)
    def _():
        pool[...] = jnp.maximum(pool[...], bm)

    @pl.when(i == GRID - 1)
    def _():
        out_ref[...] = (
            jnp.dot(pool[...], lw_ref[...], preferred_element_type=_f32)
            + lb_ref[...]
        )


def _prep_call(x, w1p, degt):
    return pl.pallas_call(
        _prep_body,
        grid=(GRID,),
        in_specs=[
            pl.BlockSpec((BN, F), lambda i: (i, 0)),
            pl.BlockSpec((F, HP), lambda i: (0, 0)),
            pl.BlockSpec((BN, NC), lambda i: (i, 0)),
        ],
        out_specs=pl.BlockSpec((BN, HP), lambda i: (i, 0)),
        out_shape=jax.ShapeDtypeStruct((NP, HP), _f32),
    )(x, w1p, degt)


def _layer_call(S, hp, degt, bp, wp):
    return pl.pallas_call(
        _layer_body,
        grid=(GRID,),
        in_specs=[
            pl.BlockSpec((NC, BN, HP), lambda i: (0, i, 0)),
            pl.BlockSpec((BN, HP), lambda i: (i, 0)),
            pl.BlockSpec((BN, NC), lambda i: (i, 0)),
            pl.BlockSpec((1, HP), lambda i: (0, 0)),
            pl.BlockSpec((HP, HP), lambda i: (0, 0)),
        ],
        out_specs=pl.BlockSpec((BN, HP), lambda i: (i, 0)),
        out_shape=jax.ShapeDtypeStruct((NP, HP), _f32),
    )(S, hp, degt, bp, wp)


def _final_call(S, hp, degt, bp, lwp, lbp):
    return pl.pallas_call(
        _final_body,
        grid=(GRID,),
        in_specs=[
            pl.BlockSpec((NC, BN, HP), lambda i: (0, i, 0)),
            pl.BlockSpec((BN, HP), lambda i: (i, 0)),
            pl.BlockSpec((BN, NC), lambda i: (i, 0)),
            pl.BlockSpec((1, HP), lambda i: (0, 0)),
            pl.BlockSpec((HP, C), lambda i: (0, 0)),
            pl.BlockSpec((1, C), lambda i: (0, 0)),
        ],
        out_specs=pl.BlockSpec((1, C), lambda i: (0, 0)),
        out_shape=jax.ShapeDtypeStruct((1, C), _f32),
        scratch_shapes=[pltpu.VMEM((1, HP), _f32)],
    )(S, hp, degt, bp, lwp, lbp)


def kernel(x, edge_index, edge_weights, batch,
           conv1_w, conv1_b, conv2_w, conv2_b, conv3_w, conv3_b,
           lin_w, lin_b):
    del batch
    x = x.astype(_f32)
    row = edge_index[0]
    col_g = edge_index[1].reshape(E // G, G)
    ew_g = edge_weights.reshape(E // G, G)

    pad_h = HP - H
    w1p = jnp.pad(conv1_w, ((0, 0), (0, pad_h)))
    w2p = jnp.pad(conv2_w, ((0, pad_h), (0, pad_h)))
    w3p = jnp.pad(conv3_w, ((0, pad_h), (0, pad_h)))
    b1p = jnp.pad(conv1_b, (0, pad_h)).reshape(1, HP)
    b2p = jnp.pad(conv2_b, (0, pad_h)).reshape(1, HP)
    b3p = jnp.pad(conv3_b, (0, pad_h)).reshape(1, HP)
    lwp = jnp.pad(lin_w, ((0, pad_h), (0, 0)))
    lbp = lin_b.reshape(1, C)

    degp = _deg_kernel(col_g, ew_g)
    degt = degp.T

    hp1 = _prep_call(x, w1p, degt)
    S1 = _agg_kernel(hp1, row, col_g, edge_weights)
    hp2 = _layer_call(S1, hp1, degt, b1p, w2p)
    S2 = _agg_kernel(hp2, row, col_g, edge_weights)
    hp3 = _layer_call(S2, hp2, degt, b2p, w3p)
    S3 = _agg_kernel(hp3, row, col_g, edge_weights)
    return _final_call(S3, hp3, degt, b3p, lwp, lbp)


# speedup vs baseline: 57.1067x; 1.2649x over previous
import functools

import jax
import jax.numpy as jnp
from jax import lax
from jax.experimental import pallas as pl
from jax.experimental.pallas import tpu as pltpu
from jax.experimental.pallas import tpu_sc as plsc

N = 10000
E = 320000
F = 128
H = 20
C = 10

HP = 32
NP = 10240
NPK = NP // 4
NC, NS = 2, 16
NW = NC * NS
G = 80
EPW = E // NW
GPW = EPW // G
CPC = 10
CH = G * CPC
NFULL = 12
TAILG = GPW - NFULL * CPC
RPT = NP // NS
EPT_D = E // NS
GPT_D = EPT_D // G
NPT = NP // NW

_mesh = plsc.VectorSubcoreMesh(core_axis_name="c", subcore_axis_name="s")
_f32 = jnp.float32
_i32 = jnp.int32
_sc_params = pltpu.CompilerParams(use_tc_tiling_on_sc=False)


@functools.partial(
    pl.kernel,
    out_type=jax.ShapeDtypeStruct((NP, HP), _f32),
    mesh=_mesh,
    compiler_params=_sc_params,
    scratch_types=[
        pltpu.VMEM_SHARED((NP,), _f32),
        pltpu.VMEM((GPT_D, G), _i32),
        pltpu.VMEM((GPT_D, G), _f32),
        pltpu.VMEM((RPT,), _f32),
        pltpu.VMEM((NPT,), _f32),
        pltpu.VMEM((NPT, HP), _f32),
        pltpu.SemaphoreType.DMA,
    ],
)
def _deg_kernel(col_hbm, ew_hbm, out_hbm, acc, coli, eww, zbuf, degv, disb, ssem):
    c = lax.axis_index("c")
    s = lax.axis_index("s")
    wid = c * NS + s

    pltpu.sync_copy(col_hbm.at[pl.ds(s * GPT_D, GPT_D)], coli)
    pltpu.sync_copy(ew_hbm.at[pl.ds(s * GPT_D, GPT_D)], eww)

    z = jnp.zeros((16,), _f32)

    def zb(i, carry):
        zbuf[pl.ds(i * 16, 16)] = z
        return carry

    lax.fori_loop(0, RPT // 16, zb, 0, unroll=8)
    pltpu.sync_copy(zbuf, acc.at[pl.ds(s * RPT, RPT)])
    plsc.subcore_barrier()

    for b in range(GPT_D // 25):
        cps = [
            pltpu.async_copy(eww.at[b * 25 + j], acc.at[coli.at[b * 25 + j]],
                             ssem, add=True)
            for j in range(25)
        ]
        for cp in cps:
            cp.wait()
    plsc.subcore_barrier()

    pltpu.sync_copy(acc.at[pl.ds(wid * NPT, NPT)], degv)

    def dis_body(i, carry):
        d = degv[pl.ds(i * 16, 16)] + 1.0
        xi = lax.bitcast_convert_type(d, _i32)
        y = lax.bitcast_convert_type(0x5F3759DF - (xi >> 1), _f32)
        for _ in range(3):
            y = y * (1.5 - 0.5 * d * y * y)
        for l in range(16):
            w = jnp.full((16,), y[l])
            disb[i * 16 + l, pl.ds(0, 16)] = w
            disb[i * 16 + l, pl.ds(16, 16)] = w
        return carry

    lax.fori_loop(0, NPT // 16, dis_body, 0)
    pltpu.sync_copy(disb, out_hbm.at[pl.ds(wid * NPT, NPT)])


@functools.partial(
    pl.kernel,
    out_type=jax.ShapeDtypeStruct((NC, NP, HP), _f32),
    mesh=_mesh,
    compiler_params=_sc_params,
    scratch_types=[
        pltpu.VMEM_SHARED((NP, HP), _f32),
        pltpu.VMEM((EPW,), _i32),
        pltpu.VMEM((GPW, G), _i32),
        pltpu.VMEM((EPW,), _f32),
        pltpu.VMEM((2, CH, HP), _f32),
        pltpu.VMEM((128, HP), _f32),
        pltpu.SemaphoreType.DMA,
        pltpu.SemaphoreType.DMA,
    ],
)
def _agg_kernel(hp_hbm, row_hbm, col_hbm, ew_hbm, out_hbm,
                acc, rowi, coli, eww, rows, zbuf, gsem, ssem):
    c = lax.axis_index("c")
    s = lax.axis_index("s")
    wid = c * NS + s

    pltpu.sync_copy(row_hbm.at[pl.ds(wid * EPW, EPW)], rowi)
    pltpu.sync_copy(col_hbm.at[pl.ds(wid * GPW, GPW)], coli)
    pltpu.sync_copy(ew_hbm.at[pl.ds(wid * EPW, EPW)], eww)

    z = jnp.zeros((16,), _f32)

    def zb(i, carry):
        zbuf[i, pl.ds(0, 16)] = z
        zbuf[i, pl.ds(16, 16)] = z
        return carry

    lax.fori_loop(0, 128, zb, 0, unroll=8)
    for q in range(RPT // 128):
        pltpu.sync_copy(zbuf, acc.at[pl.ds(s * RPT + q * 128, 128)])
    plsc.subcore_barrier()

    ngrp = [CPC] * NFULL + [TAILG]

    def fire_gathers(k):
        slot = k % 2
        g0 = k * CPC
        return [
            pltpu.async_copy(
                hp_hbm.at[rowi.at[pl.ds((g0 + gi) * G, G)]],
                rows.at[slot, pl.ds(gi * G, G)],
                gsem)
            for gi in range(ngrp[k])
        ]

    def fire_scatters(k):
        slot = k % 2
        g0 = k * CPC
        return [
            pltpu.async_copy(
                rows.at[slot, pl.ds(gi * G, G)],
                acc.at[coli.at[g0 + gi]],
                ssem, add=True)
            for gi in range(ngrp[k])
        ]

    def mul(k):
        slot = k % 2
        e0 = k * CH

        def body(i, carry):
            w16 = eww[pl.ds(e0 + i * 16, 16)]
            for l in range(16):
                e = i * 16 + l
                w = jnp.full((16,), w16[l])
                rows[slot, e, pl.ds(0, 16)] = rows[slot, e, pl.ds(0, 16)] * w
                rows[slot, e, pl.ds(16, 16)] = rows[slot, e, pl.ds(16, 16)] * w
            return carry

        lax.fori_loop(0, ngrp[k] * G // 16, body, 0)

    nchunk = NFULL + 1
    gcps = fire_gathers(0)
    scps_prev = None
    for k in range(nchunk):
        for cp in gcps:
            cp.wait()
        gcps = []
        if scps_prev is not None:
            for cp in scps_prev:
                cp.wait()
        if k + 1 < nchunk:
            gcps = fire_gathers(k + 1)
        mul(k)
        scps_prev = fire_scatters(k)
    for cp in scps_prev:
        cp.wait()

    plsc.subcore_barrier()
    for q in range(RPT // 128):
        pltpu.sync_copy(acc.at[pl.ds(s * RPT + q * 128, 128)],
                        out_hbm.at[c, pl.ds(s * RPT + q * 128, 128)])


BX = 1024
BP = BX // 4
GRID = NP // BX
NPK_REAL = N // 4


def _prep_body(x_ref, w_ref, disb_ref, out_ref):
    xr = x_ref[...].reshape(BP, 4, F)
    hs = [jnp.dot(xr[:, j, :], w_ref[...], preferred_element_type=_f32)
          for j in range(4)]
    h = jnp.concatenate(hs, axis=1)
    out_ref[...] = h * disb_ref[...]


def _layer_body(s_ref, hp_ref, disb_ref, b_ref, mseg_ref, w_ref, out_ref):
    dis = disb_ref[...]
    o = dis * (s_ref[0] + s_ref[1] + hp_ref[...]) + b_ref[...]
    s2 = jnp.dot(o * o, mseg_ref[...], preferred_element_type=_f32)
    z = jnp.maximum(o / jnp.maximum(jnp.sqrt(s2), 1e-12), 0.0)
    out_ref[...] = dis * jnp.dot(z, w_ref[...], preferred_element_type=_f32)


def _final_body(s_ref, hp_ref, disb_ref, b_ref, mseg_ref, lw_ref, lb_ref,
                out_ref, pool):
    i = pl.program_id(0)
    dis = disb_ref[...]
    o = dis * (s_ref[0] + s_ref[1] + hp_ref[...]) + b_ref[...]
    s2 = jnp.dot(o * o, mseg_ref[...], preferred_element_type=_f32)
    z = jnp.maximum(o / jnp.maximum(jnp.sqrt(s2), 1e-12), 0.0)
    rows = lax.broadcasted_iota(jnp.int32, (BP, 1), 0) + i * BP
    z = jnp.where(rows < NPK_REAL, z, 0.0)
    bm = jnp.max(z, axis=0, keepdims=True)

    @pl.when(i == 0)
    def _():
        pool[...] = bm

    @pl.when(i > 0)
    def _():
        pool[...] = jnp.maximum(pool[...], bm)

    @pl.when(i == GRID - 1)
    def _():
        p = pool[...]
        pooled = jnp.maximum(
            jnp.maximum(p[:, 0:HP], p[:, HP:2 * HP]),
            jnp.maximum(p[:, 2 * HP:3 * HP], p[:, 3 * HP:4 * HP]),
        )
        out_ref[...] = (
            jnp.dot(pooled, lw_ref[...], preferred_element_type=_f32)
            + lb_ref[...]
        )


def _prep_call(x, w1p, disb):
    return pl.pallas_call(
        _prep_body,
        grid=(GRID,),
        in_specs=[
            pl.BlockSpec((BX, F), lambda i: (i, 0)),
            pl.BlockSpec((F, HP), lambda i: (0, 0)),
            pl.BlockSpec((BP, 4 * HP), lambda i: (i, 0)),
        ],
        out_specs=pl.BlockSpec((BP, 4 * HP), lambda i: (i, 0)),
        out_shape=jax.ShapeDtypeStruct((NPK, 4 * HP), _f32),
    )(x, w1p, disb)


def _layer_call(S, hp, disb, bp, mseg, wblk):
    return pl.pallas_call(
        _layer_body,
        grid=(GRID,),
        in_specs=[
            pl.BlockSpec((NC, BP, 4 * HP), lambda i: (0, i, 0)),
            pl.BlockSpec((BP, 4 * HP), lambda i: (i, 0)),
            pl.BlockSpec((BP, 4 * HP), lambda i: (i, 0)),
            pl.BlockSpec((1, 4 * HP), lambda i: (0, 0)),
            pl.BlockSpec((4 * HP, 4 * HP), lambda i: (0, 0)),
            pl.BlockSpec((4 * HP, 4 * HP), lambda i: (0, 0)),
        ],
        out_specs=pl.BlockSpec((BP, 4 * HP), lambda i: (i, 0)),
        out_shape=jax.ShapeDtypeStruct((NPK, 4 * HP), _f32),
    )(S, hp, disb, bp, mseg, wblk)


def _final_call(S, hp, disb, bp, mseg, lwp, lbp):
    return pl.pallas_call(
        _final_body,
        grid=(GRID,),
        in_specs=[
            pl.BlockSpec((NC, BP, 4 * HP), lambda i: (0, i, 0)),
            pl.BlockSpec((BP, 4 * HP), lambda i: (i, 0)),
            pl.BlockSpec((BP, 4 * HP), lambda i: (i, 0)),
            pl.BlockSpec((1, 4 * HP), lambda i: (0, 0)),
            pl.BlockSpec((4 * HP, 4 * HP), lambda i: (0, 0)),
            pl.BlockSpec((HP, C), lambda i: (0, 0)),
            pl.BlockSpec((1, C), lambda i: (0, 0)),
        ],
        out_specs=pl.BlockSpec((1, C), lambda i: (0, 0)),
        out_shape=jax.ShapeDtypeStruct((1, C), _f32),
        scratch_shapes=[pltpu.VMEM((1, 4 * HP), _f32)],
    )(S, hp, disb, bp, mseg, lwp, lbp)


def kernel(x, edge_index, edge_weights, batch,
           conv1_w, conv1_b, conv2_w, conv2_b, conv3_w, conv3_b,
           lin_w, lin_b):
    del batch
    x = jnp.pad(x.astype(_f32), ((0, NP - N), (0, 0)))
    row = edge_index[0]
    col_g = edge_index[1].reshape(E // G, G)
    ew_g = edge_weights.reshape(E // G, G)

    pad_h = HP - H
    eye4 = jnp.eye(4, dtype=_f32)
    w1p = jnp.pad(conv1_w, ((0, 0), (0, pad_h)))
    w2b = jnp.kron(eye4, jnp.pad(conv2_w, ((0, pad_h), (0, pad_h))))
    w3b = jnp.kron(eye4, jnp.pad(conv3_w, ((0, pad_h), (0, pad_h))))
    mseg = jnp.kron(eye4, jnp.ones((HP, HP), _f32))
    b1p = jnp.tile(jnp.pad(conv1_b, (0, pad_h)), 4).reshape(1, 4 * HP)
    b2p = jnp.tile(jnp.pad(conv2_b, (0, pad_h)), 4).reshape(1, 4 * HP)
    b3p = jnp.tile(jnp.pad(conv3_b, (0, pad_h)), 4).reshape(1, 4 * HP)
    lwp = jnp.pad(lin_w, ((0, pad_h), (0, 0)))
    lbp = lin_b.reshape(1, C)

    disb = _deg_kernel(col_g, ew_g)
    disb_pk = disb.reshape(NPK, 4 * HP)

    hp1 = _prep_call(x, w1p, disb_pk)
    S1 = _agg_kernel(hp1.reshape(NP, HP), row, col_g, edge_weights)
    hp2 = _layer_call(S1.reshape(NC, NPK, 4 * HP), hp1, disb_pk, b1p, mseg, w2b)
    S2 = _agg_kernel(hp2.reshape(NP, HP), row, col_g, edge_weights)
    hp3 = _layer_call(S2.reshape(NC, NPK, 4 * HP), hp2, disb_pk, b2p, mseg, w3b)
    S3 = _agg_kernel(hp3.reshape(NP, HP), row, col_g, edge_weights)
    return _final_call(S3.reshape(NC, NPK, 4 * HP), hp3, disb_pk, b3p, mseg,
                       lwp, lbp)


# speedup vs baseline: 58.9640x; 1.0325x over previous
import functools

import jax
import jax.numpy as jnp
from jax import lax
from jax.experimental import pallas as pl
from jax.experimental.pallas import tpu as pltpu
from jax.experimental.pallas import tpu_sc as plsc

N = 10000
E = 320000
F = 128
H = 20
C = 10

HP = 32
NP = 10240
NPK = NP // 4
NC, NS = 2, 16
NW = NC * NS
G = 80
EPW = E // NW
GPW = EPW // G
CPC = 10
CH = G * CPC
NFULL = 12
TAILG = GPW - NFULL * CPC
RPT = NP // NS
EPT_D = E // NS
GPT_D = EPT_D // G
NPT = NP // NW

_mesh = plsc.VectorSubcoreMesh(core_axis_name="c", subcore_axis_name="s")
_f32 = jnp.float32
_i32 = jnp.int32
_sc_params = pltpu.CompilerParams(use_tc_tiling_on_sc=False)


@functools.partial(
    pl.kernel,
    out_type=jax.ShapeDtypeStruct((NP, HP), _f32),
    mesh=_mesh,
    compiler_params=_sc_params,
    scratch_types=[
        pltpu.VMEM_SHARED((NP,), _f32),
        pltpu.VMEM((GPT_D, G), _i32),
        pltpu.VMEM((GPT_D, G), _f32),
        pltpu.VMEM((RPT,), _f32),
        pltpu.VMEM((NPT,), _f32),
        pltpu.VMEM((NPT, HP), _f32),
        pltpu.SemaphoreType.DMA,
    ],
)
def _deg_kernel(col_hbm, ew_hbm, out_hbm, acc, coli, eww, zbuf, degv, disb, ssem):
    c = lax.axis_index("c")
    s = lax.axis_index("s")
    wid = c * NS + s

    pltpu.sync_copy(col_hbm.at[pl.ds(s * GPT_D, GPT_D)], coli)
    pltpu.sync_copy(ew_hbm.at[pl.ds(s * GPT_D, GPT_D)], eww)

    z = jnp.zeros((16,), _f32)

    def zb(i, carry):
        zbuf[pl.ds(i * 16, 16)] = z
        return carry

    lax.fori_loop(0, RPT // 16, zb, 0, unroll=8)
    pltpu.sync_copy(zbuf, acc.at[pl.ds(s * RPT, RPT)])
    plsc.subcore_barrier()

    for b in range(GPT_D // 25):
        cps = [
            pltpu.async_copy(eww.at[b * 25 + j], acc.at[coli.at[b * 25 + j]],
                             ssem, add=True)
            for j in range(25)
        ]
        for cp in cps:
            cp.wait()
    plsc.subcore_barrier()

    pltpu.sync_copy(acc.at[pl.ds(wid * NPT, NPT)], degv)

    def dis_body(i, carry):
        d = degv[pl.ds(i * 16, 16)] + 1.0
        xi = lax.bitcast_convert_type(d, _i32)
        y = lax.bitcast_convert_type(0x5F3759DF - (xi >> 1), _f32)
        for _ in range(3):
            y = y * (1.5 - 0.5 * d * y * y)
        for l in range(16):
            w = jnp.full((16,), y[l])
            disb[i * 16 + l, pl.ds(0, 16)] = w
            disb[i * 16 + l, pl.ds(16, 16)] = w
        return carry

    lax.fori_loop(0, NPT // 16, dis_body, 0)
    pltpu.sync_copy(disb, out_hbm.at[pl.ds(wid * NPT, NPT)])


@functools.partial(
    pl.kernel,
    out_type=jax.ShapeDtypeStruct((NC, NP, HP), _f32),
    mesh=_mesh,
    compiler_params=_sc_params,
    scratch_types=[
        pltpu.VMEM_SHARED((NP, HP), _f32),
        pltpu.VMEM((EPW,), _i32),
        pltpu.VMEM((GPW, G), _i32),
        pltpu.VMEM((EPW,), _f32),
        pltpu.VMEM((3, CH, HP), _f32),
        pltpu.VMEM((64, HP), _f32),
        pltpu.SemaphoreType.DMA,
        pltpu.SemaphoreType.DMA,
        pltpu.SemaphoreType.DMA,
        pltpu.SemaphoreType.DMA,
        pltpu.SemaphoreType.DMA,
        pltpu.SemaphoreType.DMA,
    ],
)
def _agg_kernel(hp_hbm, row_hbm, col_hbm, ew_hbm, out_hbm,
                acc, rowi, coli, eww, rows, zbuf,
                gsem0, gsem1, gsem2, ssem0, ssem1, ssem2):
    gsems = (gsem0, gsem1, gsem2)
    ssems = (ssem0, ssem1, ssem2)
    c = lax.axis_index("c")
    s = lax.axis_index("s")
    wid = c * NS + s

    pltpu.sync_copy(row_hbm.at[pl.ds(wid * EPW, EPW)], rowi)
    pltpu.sync_copy(col_hbm.at[pl.ds(wid * GPW, GPW)], coli)
    pltpu.sync_copy(ew_hbm.at[pl.ds(wid * EPW, EPW)], eww)

    z = jnp.zeros((16,), _f32)

    def zb(i, carry):
        zbuf[i, pl.ds(0, 16)] = z
        zbuf[i, pl.ds(16, 16)] = z
        return carry

    lax.fori_loop(0, 64, zb, 0, unroll=8)
    for q in range(RPT // 64):
        pltpu.sync_copy(zbuf, acc.at[pl.ds(s * RPT + q * 64, 64)])
    plsc.subcore_barrier()

    ngrp = [CPC] * NFULL + [TAILG]

    def fire_gathers(k):
        slot = k % 3
        g0 = k * CPC
        return [
            pltpu.async_copy(
                hp_hbm.at[rowi.at[pl.ds((g0 + gi) * G, G)]],
                rows.at[slot, pl.ds(gi * G, G)],
                gsems[slot])
            for gi in range(ngrp[k])
        ]

    def fire_scatters(k):
        slot = k % 3
        g0 = k * CPC
        return [
            pltpu.async_copy(
                rows.at[slot, pl.ds(gi * G, G)],
                acc.at[coli.at[g0 + gi]],
                ssems[slot], add=True)
            for gi in range(ngrp[k])
        ]

    def mul(k):
        slot = k % 3
        e0 = k * CH

        def body(i, carry):
            w16 = eww[pl.ds(e0 + i * 16, 16)]
            for l in range(16):
                e = i * 16 + l
                w = jnp.full((16,), w16[l])
                rows[slot, e, pl.ds(0, 16)] = rows[slot, e, pl.ds(0, 16)] * w
                rows[slot, e, pl.ds(16, 16)] = rows[slot, e, pl.ds(16, 16)] * w
            return carry

        lax.fori_loop(0, ngrp[k] * G // 16, body, 0)

    nchunk = NFULL + 1
    gcps = {0: fire_gathers(0), 1: fire_gathers(1)}
    scps = {}
    for k in range(nchunk):
        for cp in gcps.pop(k):
            cp.wait()
        mul(k)
        if k + 2 < nchunk:
            if k - 1 in scps:
                for cp in scps.pop(k - 1):
                    cp.wait()
            gcps[k + 2] = fire_gathers(k + 2)
        scps[k] = fire_scatters(k)
    for key in sorted(scps):
        for cp in scps[key]:
            cp.wait()

    plsc.subcore_barrier()
    for q in range(RPT // 320):
        pltpu.sync_copy(acc.at[pl.ds(s * RPT + q * 320, 320)],
                        out_hbm.at[c, pl.ds(s * RPT + q * 320, 320)])


BX = 1024
BP = BX // 4
GRID = NP // BX
NPK_REAL = N // 4


def _prep_body(x_ref, w_ref, disb_ref, out_ref):
    xr = x_ref[...].reshape(BP, 4, F)
    hs = [jnp.dot(xr[:, j, :], w_ref[...], preferred_element_type=_f32)
          for j in range(4)]
    h = jnp.concatenate(hs, axis=1)
    out_ref[...] = h * disb_ref[...]


def _layer_body(s_ref, hp_ref, disb_ref, b_ref, mseg_ref, w_ref, out_ref):
    dis = disb_ref[...]
    o = dis * (s_ref[0] + s_ref[1] + hp_ref[...]) + b_ref[...]
    s2 = jnp.dot(o * o, mseg_ref[...], preferred_element_type=_f32,
                 precision=lax.Precision.HIGHEST)
    z = jnp.maximum(o / jnp.maximum(jnp.sqrt(s2), 1e-12), 0.0)
    out_ref[...] = dis * jnp.dot(z, w_ref[...], preferred_element_type=_f32)


def _final_body(s_ref, hp_ref, disb_ref, b_ref, mseg_ref, lw_ref, lb_ref,
                out_ref, pool):
    i = pl.program_id(0)
    dis = disb_ref[...]
    o = dis * (s_ref[0] + s_ref[1] + hp_ref[...]) + b_ref[...]
    s2 = jnp.dot(o * o, mseg_ref[...], preferred_element_type=_f32,
                 precision=lax.Precision.HIGHEST)
    z = jnp.maximum(o / jnp.maximum(jnp.sqrt(s2), 1e-12), 0.0)
    rows = lax.broadcasted_iota(jnp.int32, (BP, 1), 0) + i * BP
    z = jnp.where(rows < NPK_REAL, z, 0.0)
    bm = jnp.max(z, axis=0, keepdims=True)

    @pl.when(i == 0)
    def _():
        pool[...] = bm

    @pl.when(i > 0)
    def _():
        pool[...] = jnp.maximum(pool[...], bm)

    @pl.when(i == GRID - 1)
    def _():
        p = pool[...]
        pooled = jnp.maximum(
            jnp.maximum(p[:, 0:HP], p[:, HP:2 * HP]),
            jnp.maximum(p[:, 2 * HP:3 * HP], p[:, 3 * HP:4 * HP]),
        )
        out_ref[...] = (
            jnp.dot(pooled, lw_ref[...], preferred_element_type=_f32)
            + lb_ref[...]
        )


def _prep_call(x, w1p, disb):
    return pl.pallas_call(
        _prep_body,
        grid=(GRID,),
        in_specs=[
            pl.BlockSpec((BX, F), lambda i: (i, 0)),
            pl.BlockSpec((F, HP), lambda i: (0, 0)),
            pl.BlockSpec((BP, 4 * HP), lambda i: (i, 0)),
        ],
        out_specs=pl.BlockSpec((BP, 4 * HP), lambda i: (i, 0)),
        out_shape=jax.ShapeDtypeStruct((NPK, 4 * HP), _f32),
    )(x, w1p, disb)


def _layer_call(S, hp, disb, bp, mseg, wblk):
    return pl.pallas_call(
        _layer_body,
        grid=(GRID,),
        in_specs=[
            pl.BlockSpec((NC, BP, 4 * HP), lambda i: (0, i, 0)),
            pl.BlockSpec((BP, 4 * HP), lambda i: (i, 0)),
            pl.BlockSpec((BP, 4 * HP), lambda i: (i, 0)),
            pl.BlockSpec((1, 4 * HP), lambda i: (0, 0)),
            pl.BlockSpec((4 * HP, 4 * HP), lambda i: (0, 0)),
            pl.BlockSpec((4 * HP, 4 * HP), lambda i: (0, 0)),
        ],
        out_specs=pl.BlockSpec((BP, 4 * HP), lambda i: (i, 0)),
        out_shape=jax.ShapeDtypeStruct((NPK, 4 * HP), _f32),
    )(S, hp, disb, bp, mseg, wblk)


def _final_call(S, hp, disb, bp, mseg, lwp, lbp):
    return pl.pallas_call(
        _final_body,
        grid=(GRID,),
        in_specs=[
            pl.BlockSpec((NC, BP, 4 * HP), lambda i: (0, i, 0)),
            pl.BlockSpec((BP, 4 * HP), lambda i: (i, 0)),
            pl.BlockSpec((BP, 4 * HP), lambda i: (i, 0)),
            pl.BlockSpec((1, 4 * HP), lambda i: (0, 0)),
            pl.BlockSpec((4 * HP, 4 * HP), lambda i: (0, 0)),
            pl.BlockSpec((HP, C), lambda i: (0, 0)),
            pl.BlockSpec((1, C), lambda i: (0, 0)),
        ],
        out_specs=pl.BlockSpec((1, C), lambda i: (0, 0)),
        out_shape=jax.ShapeDtypeStruct((1, C), _f32),
        scratch_shapes=[pltpu.VMEM((1, 4 * HP), _f32)],
    )(S, hp, disb, bp, mseg, lwp, lbp)


def kernel(x, edge_index, edge_weights, batch,
           conv1_w, conv1_b, conv2_w, conv2_b, conv3_w, conv3_b,
           lin_w, lin_b):
    del batch
    x = jnp.pad(x.astype(_f32), ((0, NP - N), (0, 0)))
    row = edge_index[0]
    col_g = edge_index[1].reshape(E // G, G)
    ew_g = edge_weights.reshape(E // G, G)

    pad_h = HP - H
    eye4 = jnp.eye(4, dtype=_f32)
    w1p = jnp.pad(conv1_w, ((0, 0), (0, pad_h)))
    w2b = jnp.kron(eye4, jnp.pad(conv2_w, ((0, pad_h), (0, pad_h))))
    w3b = jnp.kron(eye4, jnp.pad(conv3_w, ((0, pad_h), (0, pad_h))))
    mseg = jnp.kron(eye4, jnp.ones((HP, HP), _f32))
    b1p = jnp.tile(jnp.pad(conv1_b, (0, pad_h)), 4).reshape(1, 4 * HP)
    b2p = jnp.tile(jnp.pad(conv2_b, (0, pad_h)), 4).reshape(1, 4 * HP)
    b3p = jnp.tile(jnp.pad(conv3_b, (0, pad_h)), 4).reshape(1, 4 * HP)
    lwp = jnp.pad(lin_w, ((0, pad_h), (0, 0)))
    lbp = lin_b.reshape(1, C)

    disb = _deg_kernel(col_g, ew_g)
    disb_pk = disb.reshape(NPK, 4 * HP)

    hp1 = _prep_call(x, w1p, disb_pk)
    S1 = _agg_kernel(hp1.reshape(NP, HP), row, col_g, edge_weights)
    hp2 = _layer_call(S1.reshape(NC, NPK, 4 * HP), hp1, disb_pk, b1p, mseg, w2b)
    S2 = _agg_kernel(hp2.reshape(NP, HP), row, col_g, edge_weights)
    hp3 = _layer_call(S2.reshape(NC, NPK, 4 * HP), hp2, disb_pk, b2p, mseg, w3b)
    S3 = _agg_kernel(hp3.reshape(NP, HP), row, col_g, edge_weights)
    return _final_call(S3.reshape(NC, NPK, 4 * HP), hp3, disb_pk, b3p, mseg,
                       lwp, lbp)
